# Initial kernel scaffold; baseline (speedup 1.0000x reference)
import jax
import jax.numpy as jnp
from jax.experimental import pallas as pl


def kernel(pred_heatmap, pred_regression):
    raise NotImplementedError("write your pallas kernel here")


# speedup vs baseline: 5.9076x; 5.9076x over previous
import math

import jax
import jax.numpy as jnp
from jax import lax
from jax.experimental import pallas as pl
from jax.experimental.pallas import tpu as pltpu
from jax.experimental.pallas import tpu_sc as plsc

_B, _C, _H, _W = 4, 3, 96, 320
_HW = _H * _W
_ROWS = _C * _H
_TR = _ROWS // 8
_TILE_N = _TR * _W
_K = 50
_NEG = -1e30
_PI = math.pi


def _vwhere(pred, a, b):
    return jnp.where(jnp.broadcast_to(pred, a.shape), a, b)


def _vmax16(x):
    i16 = lax.iota(jnp.int32, 16)
    dn = lax.GatherDimensionNumbers(
        offset_dims=(), collapsed_slice_dims=(0,), start_index_map=(0,))
    for s in (8, 4, 2, 1):
        perm = lax.gather(x, (i16 ^ s)[:, None], dn, (1,),
                          mode=lax.GatherScatterMode.PROMISE_IN_BOUNDS)
        x = jnp.maximum(x, perm)
    return x


def _sld(ref, idx):
    return plsc.load_gather(
        ref, [jnp.broadcast_to(jnp.asarray(idx, jnp.int32), (16,))])[0]


def _sst(ref, idxs, val):
    i16 = lax.iota(jnp.int32, 16)
    plsc.store_scatter(
        ref,
        [jnp.broadcast_to(jnp.asarray(i, jnp.int32), (16,)) for i in idxs],
        jnp.broadcast_to(val, (16,)),
        mask=i16 == 0)


def _sc_body(heat_hbm, reg_hbm, pois_hbm,
             hbuf, vrow, vals, mref, smref, myv, myf,
             candv, candf, m2, selv, selfl, gsc, gfl, gidx, gbuf, rowbuf,
             shv, shf, shss, shsf, sem):
    cax = lax.axis_index("c")
    sax = lax.axis_index("s")
    b = cax * 2 + sax // 8
    bl = sax // 8
    g = sax % 8
    iota = lax.iota(jnp.int32, 16)
    neg = jnp.full((16,), _NEG, jnp.float32)

    base_row = b * _ROWS
    r0 = g * _TR
    pltpu.sync_copy(heat_hbm.at[pl.ds((base_row + r0) * _W, 38 * _W)], hbuf)

    vrow[pl.ds(0, 16)] = neg
    vrow[pl.ds(336, 16)] = neg

    def nms_row(i, carry):
        r = r0 + i
        rm = lax.rem(r, _H)
        up_ok = rm != 0
        dn_ok = rm != (_H - 1)
        for j in range(20):
            up = hbuf[pl.ds(i * _W + j * 16, 16)]
            ce = hbuf[pl.ds((i + 1) * _W + j * 16, 16)]
            dn = hbuf[pl.ds((i + 2) * _W + j * 16, 16)]
            u = _vwhere(up_ok, up, neg)
            d = _vwhere(dn_ok, dn, neg)
            vrow[pl.ds(16 + j * 16, 16)] = jnp.maximum(jnp.maximum(u, d), ce)
        for j in range(20):
            hm = jnp.maximum(
                jnp.maximum(vrow[pl.ds(15 + j * 16, 16)],
                            vrow[pl.ds(16 + j * 16, 16)]),
                vrow[pl.ds(17 + j * 16, 16)])
            ce = hbuf[pl.ds((i + 1) * _W + j * 16, 16)]
            v = jnp.where(hm == ce, ce, 0.0)
            vals[pl.ds(i * _W + j * 16, 16)] = v
            _sst(mref, [i * 20 + j], _vmax16(v))
        return carry

    lax.fori_loop(0, _TR, nms_row, None)

    smref[pl.ds(32, 16)] = neg
    for t in range(45):
        _sst(smref, [t], _vmax16(mref[pl.ds(t * 16, 16)]))

    for j in range(8):
        myv[pl.ds(j * 16, 16)] = jnp.full((16,), -1.0, jnp.float32)
        myf[pl.ds(j * 16, 16)] = jnp.zeros((16,), jnp.int32)

    def sel_body(k, carry):
        s0 = smref[pl.ds(0, 16)]
        s1 = smref[pl.ds(16, 16)]
        s2 = smref[pl.ds(32, 16)]
        m = _vmax16(jnp.maximum(jnp.maximum(s0, s1), s2))
        f0 = plsc.all_reduce_ffs(s0 == m)[0]
        f1 = plsc.all_reduce_ffs(s1 == m)[0]
        f2 = plsc.all_reduce_ffs(s2 == m)[0]
        t = jnp.where(f0 < 16, f0, jnp.where(f1 < 16, f1 + 16, f2 + 32))
        mv = plsc.load_gather(mref, [t * 16 + iota])
        l1 = plsc.all_reduce_ffs(mv == m)[0]
        v = t * 16 + l1
        vi = v // 20
        vj = v - vi * 20
        base = vi * _W + vj * 16
        vv = plsc.load_gather(vals, [base + iota])
        lane = plsc.all_reduce_ffs(vv == m)[0]
        _sst(myv, [k], m)
        _sst(myf, [k], g * _TILE_N + base + lane)
        _sst(vals, [base + lane], jnp.float32(-1.0))
        _sst(mref, [v], _vmax16(plsc.load_gather(vals, [base + iota])))
        _sst(smref, [t], _vmax16(plsc.load_gather(mref, [t * 16 + iota])))
        return carry

    lax.fori_loop(0, _K, sel_body, None)

    pltpu.sync_copy(myv, shv.at[pl.ds((bl * 8 + g) * 128, 128)])
    pltpu.sync_copy(myf, shf.at[pl.ds((bl * 8 + g) * 128, 128)])
    plsc.subcore_barrier()

    @pl.when(g == 0)
    def _():
        pltpu.sync_copy(shv.at[pl.ds(bl * 1024, 1024)], candv)
        pltpu.sync_copy(shf.at[pl.ds(bl * 1024, 1024)], candf)
        for t in range(64):
            _sst(m2, [t], _vmax16(candv[pl.ds(t * 16, 16)]))
        for j in range(8):
            selv[pl.ds(j * 16, 16)] = jnp.zeros((16,), jnp.float32)
            selfl[pl.ds(j * 16, 16)] = jnp.zeros((16,), jnp.int32)

        def msel(k, carry):
            s0 = m2[pl.ds(0, 16)]
            s1 = m2[pl.ds(16, 16)]
            s2 = m2[pl.ds(32, 16)]
            s3 = m2[pl.ds(48, 16)]
            m = _vmax16(jnp.maximum(jnp.maximum(s0, s1),
                                    jnp.maximum(s2, s3)))
            f0 = plsc.all_reduce_ffs(s0 == m)[0]
            f1 = plsc.all_reduce_ffs(s1 == m)[0]
            f2 = plsc.all_reduce_ffs(s2 == m)[0]
            f3 = plsc.all_reduce_ffs(s3 == m)[0]
            t = jnp.where(
                f0 < 16, f0,
                jnp.where(f1 < 16, f1 + 16,
                          jnp.where(f2 < 16, f2 + 32, f3 + 48)))
            vv = plsc.load_gather(candv, [t * 16 + iota])
            lane = plsc.all_reduce_ffs(vv == m)[0]
            p = t * 16 + lane
            _sst(selv, [k], m)
            _sst(selfl, [k], _sld(candf, p))
            _sst(candv, [p], jnp.float32(-1.0))
            _sst(m2, [t], _vmax16(plsc.load_gather(candv, [t * 16 + iota])))
            return carry

        lax.fori_loop(0, _K, msel, None)
        pltpu.sync_copy(selv, shss.at[pl.ds(bl * 128, 128)])
        pltpu.sync_copy(selfl, shsf.at[pl.ds(bl * 128, 128)])

    plsc.subcore_barrier()

    pltpu.sync_copy(shss.at[pl.ds(bl * 128, 128)], gsc)
    pltpu.sync_copy(shsf.at[pl.ds(bl * 128, 128)], gfl)
    k0 = g * 8
    for dl in range(8):
        k = jnp.minimum(k0 + dl, _K - 1)
        f = _sld(gfl, k)
        s = _sld(gsc, k)
        cls = (f >= _HW).astype(jnp.int32) + (f >= 2 * _HW).astype(jnp.int32)
        ind = f - cls * _HW
        rowbase = b * 50 * (_HW // 16) + ind // 16
        for j in range(4):
            cc = jnp.minimum(j * 16 + iota, 49)
            gidx[pl.ds(j * 16, 16)] = rowbase + cc * (_HW // 16)
        pltpu.async_copy(reg_hbm.at[gidx], gbuf, sem).wait()
        lanev = jnp.broadcast_to(ind - (ind // 16) * 16, (16,))
        for j in range(3):
            cv = j * 16 + iota
            rowbuf[pl.ds(dl * 64 + j * 16, 16)] = plsc.load_gather(
                gbuf, [cv, lanev])
        ys = ind // _W
        xs = ind - ys * _W
        v3 = plsc.load_gather(gbuf, [48 + iota, lanev])
        v3 = jnp.where(iota == 2, s, v3)
        v3 = jnp.where(iota == 3, cls.astype(jnp.float32), v3)
        v3 = jnp.where(iota == 4, xs.astype(jnp.float32), v3)
        v3 = jnp.where(iota == 5, ys.astype(jnp.float32), v3)
        rowbuf[pl.ds(dl * 64 + 48, 16)] = v3

    pltpu.sync_copy(rowbuf, pois_hbm.at[pl.ds((b * 64 + k0) * 64, 512)])


_sc_call = pl.kernel(
    _sc_body,
    out_type=jax.ShapeDtypeStruct((16384,), jnp.float32),
    mesh=plsc.VectorSubcoreMesh(core_axis_name="c", subcore_axis_name="s"),
    compiler_params=pltpu.CompilerParams(
        needs_layout_passes=False, use_tc_tiling_on_sc=False),
    scratch_types=[
        pltpu.VMEM((38 * 320,), jnp.float32),
        pltpu.VMEM((352,), jnp.float32),
        pltpu.VMEM((36 * 320,), jnp.float32),
        pltpu.VMEM((720,), jnp.float32),
        pltpu.VMEM((48,), jnp.float32),
        pltpu.VMEM((128,), jnp.float32),
        pltpu.VMEM((128,), jnp.int32),
        pltpu.VMEM((1024,), jnp.float32),
        pltpu.VMEM((1024,), jnp.int32),
        pltpu.VMEM((64,), jnp.float32),
        pltpu.VMEM((128,), jnp.float32),
        pltpu.VMEM((128,), jnp.int32),
        pltpu.VMEM((128,), jnp.float32),
        pltpu.VMEM((128,), jnp.int32),
        pltpu.VMEM((64,), jnp.int32),
        pltpu.VMEM((64, 16), jnp.float32),
        pltpu.VMEM((512,), jnp.float32),
        pltpu.VMEM_SHARED((2048,), jnp.float32),
        pltpu.VMEM_SHARED((2048,), jnp.int32),
        pltpu.VMEM_SHARED((256,), jnp.float32),
        pltpu.VMEM_SHARED((256,), jnp.int32),
        pltpu.SemaphoreType.DMA,
    ],
)


def _atan2(y, x):
    ax = jnp.abs(x)
    ay = jnp.abs(y)
    swap = ay > ax
    num = jnp.where(swap, ax, ay)
    den = jnp.where(swap, ay, ax)
    t = num / jnp.maximum(den, 1e-30)
    red = t > 0.4142135623730950488
    z = jnp.where(red, (t - 1.0) / (t + 1.0), t)
    z2 = z * z
    pp = ((8.05374449538e-2 * z2 - 1.38776856032e-1) * z2
          + 1.99777106478e-1) * z2 - 3.33329491539e-1
    r = z + z * z2 * pp
    r = jnp.where(red, r + 0.7853981633974483, r)
    r = jnp.where(swap, 1.5707963267948966 - r, r)
    r = jnp.where(x < 0.0, _PI - r, r)
    return jnp.where(y < 0.0, -r, r)


def _wrapf(a):
    m = a + _PI
    m = m - (2.0 * _PI) * jnp.floor(m / (2.0 * _PI))
    return m - _PI


def _dec_body(p_ref, o_ref):
    p = p_ref[...]

    def col(i):
        return lax.slice(p, (0, i), (256, i + 1))

    score = col(50)
    clsf = col(51)
    xs = col(52)
    ys = col(53)
    valid = (score >= 0.2).astype(jnp.float32)

    p0 = jnp.maximum(col(0), 0.0)
    p1 = jnp.maximum(col(1), 0.0)
    p2 = jnp.maximum(col(2), 0.0)
    p3 = jnp.maximum(col(3), 0.0)
    x1 = jnp.clip((xs - p0) * 4.0, 0.0, 1279.0)
    y1 = jnp.clip((ys - p1) * 4.0, 0.0, 383.0)
    x2 = jnp.clip((xs + p2) * 4.0, 0.0, 1279.0)
    y2 = jnp.clip((ys + p3) * 4.0, 0.0, 383.0)

    is1 = clsf == 1.0
    is2 = clsf == 2.0
    dm0 = jnp.where(is1, 1.76, jnp.where(is2, 1.73, 1.53))
    dm1 = jnp.where(is1, 0.66, jnp.where(is2, 0.60, 1.63))
    dm2 = jnp.where(is1, 0.84, jnp.where(is2, 1.76, 3.88))
    d0 = dm0 * jnp.exp(col(6))
    d1 = dm1 * jnp.exp(col(7))
    d2 = dm2 * jnp.exp(col(8))

    def sig(x):
        return 1.0 / (1.0 + jnp.exp(-x))

    direct = jnp.clip(1.0 / (sig(col(25)) + 1e-6) - 1.0, 0.1, 100.0)
    center_h = col(44) - col(46)
    c02 = ((col(28) - col(36)) + (col(32) - col(40))) * 0.5
    c13 = ((col(30) - col(38)) + (col(34) - col(42))) * 0.5
    h0 = jnp.maximum(center_h, 0.1)
    h1 = jnp.maximum(c02, 0.1)
    h2 = jnp.maximum(c13, 0.1)
    fh = 721.5377 * d0
    kd0 = jnp.clip(fh / (4.0 * h0), 0.1, 100.0)
    kd1 = jnp.clip(fh / (4.0 * h1), 0.1, 100.0)
    kd2 = jnp.clip(fh / (4.0 * h2), 0.1, 100.0)

    u0 = jnp.exp(col(26))
    u1 = jnp.exp(col(47))
    u2 = jnp.exp(col(48))
    u3 = jnp.exp(col(49))
    w0 = 1.0 / u0
    w1 = 1.0 / u1
    w2 = 1.0 / u2
    w3 = 1.0 / u3
    ws = w0 + w1 + w2 + w3
    depth = (direct * w0 + kd0 * w1 + kd1 * w2 + kd2 * w3) / ws

    projx = (xs + col(4)) * 4.0
    projy = (ys + col(5)) * 4.0
    x3d = (projx - 609.5593) * depth / 721.5377
    y3d = (projy - 172.854) * depth / 721.5377

    conf0 = sig(col(10) - col(9))
    conf1 = sig(col(12) - col(11))
    conf2 = sig(col(14) - col(13))
    conf3 = sig(col(16) - col(15))
    best = conf0
    binf = jnp.zeros_like(conf0)
    for i, cf in ((1.0, conf1), (2.0, conf2), (3.0, conf3)):
        upd = cf > best
        binf = jnp.where(upd, i, binf)
        best = jnp.where(upd, cf, best)
    b1 = binf == 1.0
    b2 = binf == 2.0
    b3 = binf == 3.0
    sel_s = jnp.where(b1, col(19), jnp.where(b2, col(21),
                      jnp.where(b3, col(23), col(17))))
    sel_c = jnp.where(b1, col(20), jnp.where(b2, col(22),
                      jnp.where(b3, col(24), col(18))))
    nrm = jnp.sqrt(sel_s * sel_s + sel_c * sel_c) + 1e-9
    ac = jnp.where(b1, _PI / 2.0, jnp.where(b2, _PI,
                   jnp.where(b3, -_PI / 2.0, 0.0)))
    alpha = _wrapf(_atan2(sel_s / nrm, sel_c / nrm) + ac)
    roty = _wrapf(alpha + _atan2(x3d, depth))

    out = jnp.concatenate(
        [clsf, alpha, x1, y1, x2, y2, d0, d1, d2, x3d, y3d, depth,
         roty, score], axis=1) * valid
    o_ref[...] = out


_dec_call = pl.pallas_call(
    _dec_body,
    out_shape=jax.ShapeDtypeStruct((256, 14), jnp.float32),
)


def kernel(pred_heatmap, pred_regression):
    pad = jnp.zeros((_W,), jnp.float32)
    heat1 = jnp.concatenate([pad, pred_heatmap.reshape(-1), pad])
    reg2 = pred_regression.reshape(_B * 50 * (_HW // 16), 16)
    pois = _sc_call(heat1, reg2)
    res = _dec_call(pois.reshape(256, 64))
    return res.reshape(_B, 64, 14)[:, :_K, :].reshape(_B * _K, 14)


# speedup vs baseline: 6.0651x; 1.0267x over previous
import math

import jax
import jax.numpy as jnp
from jax import lax
from jax.experimental import pallas as pl
from jax.experimental.pallas import tpu as pltpu
from jax.experimental.pallas import tpu_sc as plsc

_B, _C, _H, _W = 4, 3, 96, 320
_HW = _H * _W
_ROWS = _C * _H
_TR = _ROWS // 8
_TILE_N = _TR * _W
_K = 50
_NEG = -1e30
_PI = math.pi


def _vwhere(pred, a, b):
    return jnp.where(jnp.broadcast_to(pred, a.shape), a, b)


def _vmax16(x):
    i16 = lax.iota(jnp.int32, 16)
    dn = lax.GatherDimensionNumbers(
        offset_dims=(), collapsed_slice_dims=(0,), start_index_map=(0,))
    for s in (8, 4, 2, 1):
        perm = lax.gather(x, (i16 ^ s)[:, None], dn, (1,),
                          mode=lax.GatherScatterMode.PROMISE_IN_BOUNDS)
        x = jnp.maximum(x, perm)
    return x


def _sld(ref, idx):
    return plsc.load_gather(
        ref, [jnp.broadcast_to(jnp.asarray(idx, jnp.int32), (16,))])[0]


def _sst(ref, idxs, val):
    i16 = lax.iota(jnp.int32, 16)
    plsc.store_scatter(
        ref,
        [jnp.broadcast_to(jnp.asarray(i, jnp.int32), (16,)) for i in idxs],
        jnp.broadcast_to(val, (16,)),
        mask=i16 == 0)


def _sc_body(heat_hbm, reg_hbm, pois_hbm,
             hbuf, vrow, vals, mref, smref, myv, myf,
             candv, candf, m2, selv, selfl, gsc, gfl, gidx, gbuf, rowbuf,
             shv, shf, shss, shsf, sem):
    cax = lax.axis_index("c")
    sax = lax.axis_index("s")
    b = cax * 2 + sax // 8
    bl = sax // 8
    g = sax % 8
    iota = lax.iota(jnp.int32, 16)
    neg = jnp.full((16,), _NEG, jnp.float32)

    base_row = b * _ROWS
    r0 = g * _TR
    pltpu.sync_copy(heat_hbm.at[pl.ds((base_row + r0) * _W, 38 * _W)], hbuf)

    vrow[pl.ds(0, 16)] = neg
    vrow[pl.ds(336, 16)] = neg

    def nms_row(i, carry):
        r = r0 + i
        rm = lax.rem(r, _H)
        up_ok = rm != 0
        dn_ok = rm != (_H - 1)
        for j in range(20):
            up = hbuf[pl.ds(i * _W + j * 16, 16)]
            ce = hbuf[pl.ds((i + 1) * _W + j * 16, 16)]
            dn = hbuf[pl.ds((i + 2) * _W + j * 16, 16)]
            u = _vwhere(up_ok, up, neg)
            d = _vwhere(dn_ok, dn, neg)
            vrow[pl.ds(16 + j * 16, 16)] = jnp.maximum(jnp.maximum(u, d), ce)
        for j in range(20):
            hm = jnp.maximum(
                jnp.maximum(vrow[pl.ds(15 + j * 16, 16)],
                            vrow[pl.ds(16 + j * 16, 16)]),
                vrow[pl.ds(17 + j * 16, 16)])
            ce = hbuf[pl.ds((i + 1) * _W + j * 16, 16)]
            v = jnp.where(hm == ce, ce, 0.0)
            vals[pl.ds(i * _W + j * 16, 16)] = v
            _sst(mref, [i * 20 + j], _vmax16(v))
        return carry

    lax.fori_loop(0, _TR, nms_row, None)

    smref[pl.ds(32, 16)] = neg
    for t in range(45):
        _sst(smref, [t], _vmax16(mref[pl.ds(t * 16, 16)]))

    for j in range(8):
        myv[pl.ds(j * 16, 16)] = jnp.full((16,), -1.0, jnp.float32)
        myf[pl.ds(j * 16, 16)] = jnp.zeros((16,), jnp.int32)

    def sel_body(k, carry):
        s0 = smref[pl.ds(0, 16)]
        s1 = smref[pl.ds(16, 16)]
        s2 = smref[pl.ds(32, 16)]
        m = _vmax16(jnp.maximum(jnp.maximum(s0, s1), s2))
        f0 = plsc.all_reduce_ffs(s0 == m)[0]
        f1 = plsc.all_reduce_ffs(s1 == m)[0]
        f2 = plsc.all_reduce_ffs(s2 == m)[0]
        t = jnp.where(f0 < 16, f0, jnp.where(f1 < 16, f1 + 16, f2 + 32))
        mv = plsc.load_gather(mref, [t * 16 + iota])
        l1 = plsc.all_reduce_ffs(mv == m)[0]
        v = t * 16 + l1
        vi = v // 20
        vj = v - vi * 20
        base = vi * _W + vj * 16
        vv = plsc.load_gather(vals, [base + iota])
        lane = plsc.all_reduce_ffs(vv == m)[0]
        _sst(myv, [k], m)
        _sst(myf, [k], g * _TILE_N + base + lane)
        _sst(vals, [base + lane], jnp.float32(-1.0))
        _sst(mref, [v], _vmax16(plsc.load_gather(vals, [base + iota])))
        _sst(smref, [t], _vmax16(plsc.load_gather(mref, [t * 16 + iota])))
        return carry

    lax.fori_loop(0, _K, sel_body, None)

    pltpu.sync_copy(myv, shv.at[pl.ds((bl * 8 + g) * 128, 128)])
    pltpu.sync_copy(myf, shf.at[pl.ds((bl * 8 + g) * 128, 128)])
    plsc.subcore_barrier()

    @pl.when(g == 0)
    def _():
        pltpu.sync_copy(shv.at[pl.ds(bl * 1024, 1024)], candv)
        pltpu.sync_copy(shf.at[pl.ds(bl * 1024, 1024)], candf)
        for t in range(64):
            _sst(m2, [t], _vmax16(candv[pl.ds(t * 16, 16)]))
        for j in range(8):
            selv[pl.ds(j * 16, 16)] = jnp.zeros((16,), jnp.float32)
            selfl[pl.ds(j * 16, 16)] = jnp.zeros((16,), jnp.int32)

        def msel(k, carry):
            s0 = m2[pl.ds(0, 16)]
            s1 = m2[pl.ds(16, 16)]
            s2 = m2[pl.ds(32, 16)]
            s3 = m2[pl.ds(48, 16)]
            m = _vmax16(jnp.maximum(jnp.maximum(s0, s1),
                                    jnp.maximum(s2, s3)))
            f0 = plsc.all_reduce_ffs(s0 == m)[0]
            f1 = plsc.all_reduce_ffs(s1 == m)[0]
            f2 = plsc.all_reduce_ffs(s2 == m)[0]
            f3 = plsc.all_reduce_ffs(s3 == m)[0]
            t = jnp.where(
                f0 < 16, f0,
                jnp.where(f1 < 16, f1 + 16,
                          jnp.where(f2 < 16, f2 + 32, f3 + 48)))
            vv = plsc.load_gather(candv, [t * 16 + iota])
            lane = plsc.all_reduce_ffs(vv == m)[0]
            p = t * 16 + lane
            _sst(selv, [k], m)
            _sst(selfl, [k], _sld(candf, p))
            _sst(candv, [p], jnp.float32(-1.0))
            _sst(m2, [t], _vmax16(plsc.load_gather(candv, [t * 16 + iota])))
            return carry

        lax.fori_loop(0, _K, msel, None)
        pltpu.sync_copy(selv, shss.at[pl.ds(bl * 128, 128)])
        pltpu.sync_copy(selfl, shsf.at[pl.ds(bl * 128, 128)])

    plsc.subcore_barrier()

    pltpu.sync_copy(shss.at[pl.ds(bl * 128, 128)], gsc)
    pltpu.sync_copy(shsf.at[pl.ds(bl * 128, 128)], gfl)
    k0 = g * 8
    for dl in range(8):
        k = jnp.minimum(k0 + dl, _K - 1)
        f = _sld(gfl, k)
        cls = (f >= _HW).astype(jnp.int32) + (f >= 2 * _HW).astype(jnp.int32)
        ind = f - cls * _HW
        rowbase = b * 50 * (_HW // 16) + ind // 16
        for j in range(4):
            cc = jnp.minimum(j * 16 + iota, 49)
            gidx[dl // 2, pl.ds((dl % 2) * 64 + j * 16, 16)] = (
                rowbase + cc * (_HW // 16))
    cps = [pltpu.async_copy(reg_hbm.at[gidx.at[q]],
                            gbuf.at[pl.ds(q * 128, 128)], sem)
           for q in range(4)]
    for cp in cps:
        cp.wait()
    for dl in range(8):
        k = jnp.minimum(k0 + dl, _K - 1)
        f = _sld(gfl, k)
        s = _sld(gsc, k)
        cls = (f >= _HW).astype(jnp.int32) + (f >= 2 * _HW).astype(jnp.int32)
        ind = f - cls * _HW
        lanev = jnp.broadcast_to(ind - (ind // 16) * 16, (16,))
        for j in range(3):
            cv = dl * 64 + j * 16 + iota
            rowbuf[pl.ds(dl * 64 + j * 16, 16)] = plsc.load_gather(
                gbuf, [cv, lanev])
        ys = ind // _W
        xs = ind - ys * _W
        v3 = plsc.load_gather(gbuf, [dl * 64 + 48 + iota, lanev])
        v3 = jnp.where(iota == 2, s, v3)
        v3 = jnp.where(iota == 3, cls.astype(jnp.float32), v3)
        v3 = jnp.where(iota == 4, xs.astype(jnp.float32), v3)
        v3 = jnp.where(iota == 5, ys.astype(jnp.float32), v3)
        rowbuf[pl.ds(dl * 64 + 48, 16)] = v3

    pltpu.sync_copy(rowbuf, pois_hbm.at[pl.ds((b * 64 + k0) * 64, 512)])


_sc_call = pl.kernel(
    _sc_body,
    out_type=jax.ShapeDtypeStruct((16384,), jnp.float32),
    mesh=plsc.VectorSubcoreMesh(core_axis_name="c", subcore_axis_name="s"),
    compiler_params=pltpu.CompilerParams(
        needs_layout_passes=False, use_tc_tiling_on_sc=False),
    scratch_types=[
        pltpu.VMEM((38 * 320,), jnp.float32),
        pltpu.VMEM((352,), jnp.float32),
        pltpu.VMEM((36 * 320,), jnp.float32),
        pltpu.VMEM((720,), jnp.float32),
        pltpu.VMEM((48,), jnp.float32),
        pltpu.VMEM((128,), jnp.float32),
        pltpu.VMEM((128,), jnp.int32),
        pltpu.VMEM((1024,), jnp.float32),
        pltpu.VMEM((1024,), jnp.int32),
        pltpu.VMEM((64,), jnp.float32),
        pltpu.VMEM((128,), jnp.float32),
        pltpu.VMEM((128,), jnp.int32),
        pltpu.VMEM((128,), jnp.float32),
        pltpu.VMEM((128,), jnp.int32),
        pltpu.VMEM((4, 128), jnp.int32),
        pltpu.VMEM((512, 16), jnp.float32),
        pltpu.VMEM((512,), jnp.float32),
        pltpu.VMEM_SHARED((2048,), jnp.float32),
        pltpu.VMEM_SHARED((2048,), jnp.int32),
        pltpu.VMEM_SHARED((256,), jnp.float32),
        pltpu.VMEM_SHARED((256,), jnp.int32),
        pltpu.SemaphoreType.DMA,
    ],
)


def _atan2(y, x):
    ax = jnp.abs(x)
    ay = jnp.abs(y)
    swap = ay > ax
    num = jnp.where(swap, ax, ay)
    den = jnp.where(swap, ay, ax)
    t = num / jnp.maximum(den, 1e-30)
    red = t > 0.4142135623730950488
    z = jnp.where(red, (t - 1.0) / (t + 1.0), t)
    z2 = z * z
    pp = ((8.05374449538e-2 * z2 - 1.38776856032e-1) * z2
          + 1.99777106478e-1) * z2 - 3.33329491539e-1
    r = z + z * z2 * pp
    r = jnp.where(red, r + 0.7853981633974483, r)
    r = jnp.where(swap, 1.5707963267948966 - r, r)
    r = jnp.where(x < 0.0, _PI - r, r)
    return jnp.where(y < 0.0, -r, r)


def _wrapf(a):
    m = a + _PI
    m = m - (2.0 * _PI) * jnp.floor(m / (2.0 * _PI))
    return m - _PI


def _dec_body(p_ref, o_ref):
    p = p_ref[...]

    def col(i):
        return lax.slice(p, (0, i), (256, i + 1))

    score = col(50)
    clsf = col(51)
    xs = col(52)
    ys = col(53)
    valid = (score >= 0.2).astype(jnp.float32)

    p0 = jnp.maximum(col(0), 0.0)
    p1 = jnp.maximum(col(1), 0.0)
    p2 = jnp.maximum(col(2), 0.0)
    p3 = jnp.maximum(col(3), 0.0)
    x1 = jnp.clip((xs - p0) * 4.0, 0.0, 1279.0)
    y1 = jnp.clip((ys - p1) * 4.0, 0.0, 383.0)
    x2 = jnp.clip((xs + p2) * 4.0, 0.0, 1279.0)
    y2 = jnp.clip((ys + p3) * 4.0, 0.0, 383.0)

    is1 = clsf == 1.0
    is2 = clsf == 2.0
    dm0 = jnp.where(is1, 1.76, jnp.where(is2, 1.73, 1.53))
    dm1 = jnp.where(is1, 0.66, jnp.where(is2, 0.60, 1.63))
    dm2 = jnp.where(is1, 0.84, jnp.where(is2, 1.76, 3.88))
    d0 = dm0 * jnp.exp(col(6))
    d1 = dm1 * jnp.exp(col(7))
    d2 = dm2 * jnp.exp(col(8))

    def sig(x):
        return 1.0 / (1.0 + jnp.exp(-x))

    direct = jnp.clip(1.0 / (sig(col(25)) + 1e-6) - 1.0, 0.1, 100.0)
    center_h = col(44) - col(46)
    c02 = ((col(28) - col(36)) + (col(32) - col(40))) * 0.5
    c13 = ((col(30) - col(38)) + (col(34) - col(42))) * 0.5
    h0 = jnp.maximum(center_h, 0.1)
    h1 = jnp.maximum(c02, 0.1)
    h2 = jnp.maximum(c13, 0.1)
    fh = 721.5377 * d0
    kd0 = jnp.clip(fh / (4.0 * h0), 0.1, 100.0)
    kd1 = jnp.clip(fh / (4.0 * h1), 0.1, 100.0)
    kd2 = jnp.clip(fh / (4.0 * h2), 0.1, 100.0)

    u0 = jnp.exp(col(26))
    u1 = jnp.exp(col(47))
    u2 = jnp.exp(col(48))
    u3 = jnp.exp(col(49))
    w0 = 1.0 / u0
    w1 = 1.0 / u1
    w2 = 1.0 / u2
    w3 = 1.0 / u3
    ws = w0 + w1 + w2 + w3
    depth = (direct * w0 + kd0 * w1 + kd1 * w2 + kd2 * w3) / ws

    projx = (xs + col(4)) * 4.0
    projy = (ys + col(5)) * 4.0
    x3d = (projx - 609.5593) * depth / 721.5377
    y3d = (projy - 172.854) * depth / 721.5377

    conf0 = sig(col(10) - col(9))
    conf1 = sig(col(12) - col(11))
    conf2 = sig(col(14) - col(13))
    conf3 = sig(col(16) - col(15))
    best = conf0
    binf = jnp.zeros_like(conf0)
    for i, cf in ((1.0, conf1), (2.0, conf2), (3.0, conf3)):
        upd = cf > best
        binf = jnp.where(upd, i, binf)
        best = jnp.where(upd, cf, best)
    b1 = binf == 1.0
    b2 = binf == 2.0
    b3 = binf == 3.0
    sel_s = jnp.where(b1, col(19), jnp.where(b2, col(21),
                      jnp.where(b3, col(23), col(17))))
    sel_c = jnp.where(b1, col(20), jnp.where(b2, col(22),
                      jnp.where(b3, col(24), col(18))))
    nrm = jnp.sqrt(sel_s * sel_s + sel_c * sel_c) + 1e-9
    ac = jnp.where(b1, _PI / 2.0, jnp.where(b2, _PI,
                   jnp.where(b3, -_PI / 2.0, 0.0)))
    alpha = _wrapf(_atan2(sel_s / nrm, sel_c / nrm) + ac)
    roty = _wrapf(alpha + _atan2(x3d, depth))

    out = jnp.concatenate(
        [clsf, alpha, x1, y1, x2, y2, d0, d1, d2, x3d, y3d, depth,
         roty, score], axis=1) * valid
    o_ref[...] = out


_dec_call = pl.pallas_call(
    _dec_body,
    out_shape=jax.ShapeDtypeStruct((256, 14), jnp.float32),
)


def kernel(pred_heatmap, pred_regression):
    pad = jnp.zeros((_W,), jnp.float32)
    heat1 = jnp.concatenate([pad, pred_heatmap.reshape(-1), pad])
    reg2 = pred_regression.reshape(_B * 50 * (_HW // 16), 16)
    pois = _sc_call(heat1, reg2)
    res = _dec_call(pois.reshape(256, 64))
    return res.reshape(_B, 64, 14)[:, :_K, :].reshape(_B * _K, 14)


# speedup vs baseline: 7.1766x; 1.1833x over previous
import math

import jax
import jax.numpy as jnp
from jax import lax
from jax.experimental import pallas as pl
from jax.experimental.pallas import tpu as pltpu
from jax.experimental.pallas import tpu_sc as plsc

_B, _C, _H, _W = 4, 3, 96, 320
_HW = _H * _W
_ROWS = _C * _H
_TR = _ROWS // 8
_TILE_N = _TR * _W
_K = 50
_NEG = -1e30
_PI = math.pi


def _vwhere(pred, a, b):
    return jnp.where(jnp.broadcast_to(pred, a.shape), a, b)


def _vmax16(x):
    i16 = lax.iota(jnp.int32, 16)
    dn = lax.GatherDimensionNumbers(
        offset_dims=(), collapsed_slice_dims=(0,), start_index_map=(0,))
    for s in (8, 4, 2, 1):
        perm = lax.gather(x, (i16 ^ s)[:, None], dn, (1,),
                          mode=lax.GatherScatterMode.PROMISE_IN_BOUNDS)
        x = jnp.maximum(x, perm)
    return x


def _sld(ref, idx):
    return plsc.load_gather(
        ref, [jnp.broadcast_to(jnp.asarray(idx, jnp.int32), (16,))])[0]


def _sst(ref, idxs, val):
    i16 = lax.iota(jnp.int32, 16)
    plsc.store_scatter(
        ref,
        [jnp.broadcast_to(jnp.asarray(i, jnp.int32), (16,)) for i in idxs],
        jnp.broadcast_to(val, (16,)),
        mask=i16 == 0)


def _sc_body(heat_hbm, reg_hbm, out_hbm,
             hbuf, vrow, vals, mref, smref, myv, myf,
             candv, candf, m2, selv, selfl, gsc, gfl, gidx, gbuf, obuf,
             shv, shf, shss, shsf, sem):
    cax = lax.axis_index("c")
    sax = lax.axis_index("s")
    b = cax * 2 + sax // 8
    bl = sax // 8
    g = sax % 8
    iota = lax.iota(jnp.int32, 16)
    neg = jnp.full((16,), _NEG, jnp.float32)

    base_row = b * _ROWS
    r0 = g * _TR
    first = (b == 0) & (g == 0)

    @pl.when(first)
    def _():
        pltpu.sync_copy(heat_hbm.at[pl.ds(0, 38 * _W)],
                        hbuf.at[pl.ds(2 * _W, 38 * _W)])

    @pl.when(jnp.logical_not(first))
    def _():
        pltpu.sync_copy(
            heat_hbm.at[pl.ds((base_row + r0 - 2) * _W, 40 * _W)], hbuf)

    vrow[pl.ds(0, 16)] = neg
    vrow[pl.ds(336, 16)] = neg

    def nms_row(i, carry):
        r = r0 + i
        rm = lax.rem(r, _H)
        up_ok = rm != 0
        dn_ok = rm != (_H - 1)
        for j in range(20):
            up = hbuf[pl.ds((i + 1) * _W + j * 16, 16)]
            ce = hbuf[pl.ds((i + 2) * _W + j * 16, 16)]
            dn = hbuf[pl.ds((i + 3) * _W + j * 16, 16)]
            u = _vwhere(up_ok, up, neg)
            d = _vwhere(dn_ok, dn, neg)
            vrow[pl.ds(16 + j * 16, 16)] = jnp.maximum(jnp.maximum(u, d), ce)
        for j in range(20):
            hm = jnp.maximum(
                jnp.maximum(vrow[pl.ds(15 + j * 16, 16)],
                            vrow[pl.ds(16 + j * 16, 16)]),
                vrow[pl.ds(17 + j * 16, 16)])
            ce = hbuf[pl.ds((i + 2) * _W + j * 16, 16)]
            v = jnp.where(hm == ce, ce, 0.0)
            vals[pl.ds(i * _W + j * 16, 16)] = v
            _sst(mref, [i * 20 + j], _vmax16(v))
        return carry

    lax.fori_loop(0, _TR, nms_row, None)

    smref[pl.ds(32, 16)] = neg
    for t in range(45):
        _sst(smref, [t], _vmax16(mref[pl.ds(t * 16, 16)]))

    for j in range(8):
        myv[pl.ds(j * 16, 16)] = jnp.full((16,), -1.0, jnp.float32)
        myf[pl.ds(j * 16, 16)] = jnp.zeros((16,), jnp.int32)

    def sel_body(k, carry):
        s0 = smref[pl.ds(0, 16)]
        s1 = smref[pl.ds(16, 16)]
        s2 = smref[pl.ds(32, 16)]
        m = _vmax16(jnp.maximum(jnp.maximum(s0, s1), s2))
        f0 = plsc.all_reduce_ffs(s0 == m)[0]
        f1 = plsc.all_reduce_ffs(s1 == m)[0]
        f2 = plsc.all_reduce_ffs(s2 == m)[0]
        t = jnp.where(f0 < 16, f0, jnp.where(f1 < 16, f1 + 16, f2 + 32))
        mv = plsc.load_gather(mref, [t * 16 + iota])
        l1 = plsc.all_reduce_ffs(mv == m)[0]
        v = t * 16 + l1
        vi = v // 20
        vj = v - vi * 20
        base = vi * _W + vj * 16
        vv = plsc.load_gather(vals, [base + iota])
        lane = plsc.all_reduce_ffs(vv == m)[0]
        _sst(myv, [k], m)
        _sst(myf, [k], g * _TILE_N + base + lane)
        _sst(vals, [base + lane], jnp.float32(-1.0))
        _sst(mref, [v], _vmax16(plsc.load_gather(vals, [base + iota])))
        _sst(smref, [t], _vmax16(plsc.load_gather(mref, [t * 16 + iota])))
        return carry

    lax.fori_loop(0, _K, sel_body, None)

    pltpu.sync_copy(myv, shv.at[pl.ds((bl * 8 + g) * 128, 128)])
    pltpu.sync_copy(myf, shf.at[pl.ds((bl * 8 + g) * 128, 128)])
    plsc.subcore_barrier()

    @pl.when(g == 0)
    def _():
        pltpu.sync_copy(shv.at[pl.ds(bl * 1024, 1024)], candv)
        pltpu.sync_copy(shf.at[pl.ds(bl * 1024, 1024)], candf)
        for t in range(64):
            _sst(m2, [t], _vmax16(candv[pl.ds(t * 16, 16)]))
        for j in range(8):
            selv[pl.ds(j * 16, 16)] = jnp.zeros((16,), jnp.float32)
            selfl[pl.ds(j * 16, 16)] = jnp.zeros((16,), jnp.int32)

        def msel(k, carry):
            s0 = m2[pl.ds(0, 16)]
            s1 = m2[pl.ds(16, 16)]
            s2 = m2[pl.ds(32, 16)]
            s3 = m2[pl.ds(48, 16)]
            m = _vmax16(jnp.maximum(jnp.maximum(s0, s1),
                                    jnp.maximum(s2, s3)))
            f0 = plsc.all_reduce_ffs(s0 == m)[0]
            f1 = plsc.all_reduce_ffs(s1 == m)[0]
            f2 = plsc.all_reduce_ffs(s2 == m)[0]
            f3 = plsc.all_reduce_ffs(s3 == m)[0]
            t = jnp.where(
                f0 < 16, f0,
                jnp.where(f1 < 16, f1 + 16,
                          jnp.where(f2 < 16, f2 + 32, f3 + 48)))
            vv = plsc.load_gather(candv, [t * 16 + iota])
            lane = plsc.all_reduce_ffs(vv == m)[0]
            p = t * 16 + lane
            _sst(selv, [k], m)
            _sst(selfl, [k], _sld(candf, p))
            _sst(candv, [p], jnp.float32(-1.0))
            _sst(m2, [t], _vmax16(plsc.load_gather(candv, [t * 16 + iota])))
            return carry

        lax.fori_loop(0, _K, msel, None)
        pltpu.sync_copy(selv, shss.at[pl.ds(bl * 128, 128)])
        pltpu.sync_copy(selfl, shsf.at[pl.ds(bl * 128, 128)])

    plsc.subcore_barrier()

    pltpu.sync_copy(shss.at[pl.ds(bl * 128, 128)], gsc)
    pltpu.sync_copy(shsf.at[pl.ds(bl * 128, 128)], gfl)
    k0 = g * 8
    lane8 = lax.rem(iota, 8)
    indv = jnp.zeros((16,), jnp.int32)
    clsv = jnp.zeros((16,), jnp.int32)
    scv = jnp.zeros((16,), jnp.float32)
    for dl in range(8):
        k = jnp.minimum(k0 + dl, _K - 1)
        f = _sld(gfl, k)
        s = _sld(gsc, k)
        cls = (f >= _HW).astype(jnp.int32) + (f >= 2 * _HW).astype(jnp.int32)
        ind = f - cls * _HW
        indv = jnp.where(lane8 == dl, ind, indv)
        clsv = jnp.where(lane8 == dl, cls, clsv)
        scv = jnp.where(lane8 == dl, s, scv)
        rowbase = b * 50 * (_HW // 16) + ind // 16
        for j in range(4):
            cc = jnp.minimum(j * 16 + iota, 49)
            gidx[dl // 2, pl.ds((dl % 2) * 64 + j * 16, 16)] = (
                rowbase + cc * (_HW // 16))
    cps = [pltpu.async_copy(reg_hbm.at[gidx.at[q]],
                            gbuf.at[pl.ds(q * 128, 128)], sem)
           for q in range(4)]
    for cp in cps:
        cp.wait()

    rowsel = lane8 * 64
    lanev = indv - (indv // 16) * 16

    def chan(c):
        return plsc.load_gather(gbuf, [rowsel + c, lanev])

    score = scv
    clsf = clsv.astype(jnp.float32)
    ysi = indv // _W
    xs = (indv - ysi * _W).astype(jnp.float32)
    ys = ysi.astype(jnp.float32)
    valid = jnp.where(score >= 0.2, 1.0, 0.0)

    p0 = jnp.maximum(chan(0), 0.0)
    p1 = jnp.maximum(chan(1), 0.0)
    p2 = jnp.maximum(chan(2), 0.0)
    p3 = jnp.maximum(chan(3), 0.0)
    x1 = jnp.clip((xs - p0) * 4.0, 0.0, 1279.0)
    y1 = jnp.clip((ys - p1) * 4.0, 0.0, 383.0)
    x2 = jnp.clip((xs + p2) * 4.0, 0.0, 1279.0)
    y2 = jnp.clip((ys + p3) * 4.0, 0.0, 383.0)

    is1 = clsv == 1
    is2 = clsv == 2
    dm0 = jnp.where(is1, 1.76, jnp.where(is2, 1.73, 1.53))
    dm1 = jnp.where(is1, 0.66, jnp.where(is2, 0.60, 1.63))
    dm2 = jnp.where(is1, 0.84, jnp.where(is2, 1.76, 3.88))
    d0 = dm0 * jnp.exp(chan(6))
    d1 = dm1 * jnp.exp(chan(7))
    d2 = dm2 * jnp.exp(chan(8))

    def sig(x):
        return 1.0 / (1.0 + jnp.exp(-x))

    direct = jnp.clip(1.0 / (sig(chan(25)) + 1e-6) - 1.0, 0.1, 100.0)
    center_h = chan(44) - chan(46)
    c02 = ((chan(28) - chan(36)) + (chan(32) - chan(40))) * 0.5
    c13 = ((chan(30) - chan(38)) + (chan(34) - chan(42))) * 0.5
    h0 = jnp.maximum(center_h, 0.1)
    h1 = jnp.maximum(c02, 0.1)
    h2 = jnp.maximum(c13, 0.1)
    fh = 721.5377 * d0
    kd0 = jnp.clip(fh / (4.0 * h0), 0.1, 100.0)
    kd1 = jnp.clip(fh / (4.0 * h1), 0.1, 100.0)
    kd2 = jnp.clip(fh / (4.0 * h2), 0.1, 100.0)

    u0 = jnp.exp(chan(26))
    u1 = jnp.exp(chan(47))
    u2 = jnp.exp(chan(48))
    u3 = jnp.exp(chan(49))
    w0 = 1.0 / u0
    w1 = 1.0 / u1
    w2 = 1.0 / u2
    w3 = 1.0 / u3
    ws = w0 + w1 + w2 + w3
    depth = (direct * w0 + kd0 * w1 + kd1 * w2 + kd2 * w3) / ws

    projx = (xs + chan(4)) * 4.0
    projy = (ys + chan(5)) * 4.0
    x3d = (projx - 609.5593) * depth / 721.5377
    y3d = (projy - 172.854) * depth / 721.5377

    conf0 = sig(chan(10) - chan(9))
    conf1 = sig(chan(12) - chan(11))
    conf2 = sig(chan(14) - chan(13))
    conf3 = sig(chan(16) - chan(15))
    best = conf0
    binf = jnp.zeros((16,), jnp.int32)
    for i, cf in ((1, conf1), (2, conf2), (3, conf3)):
        upd = cf > best
        binf = jnp.where(upd, i, binf)
        best = jnp.where(upd, cf, best)
    b1 = binf == 1
    b2 = binf == 2
    b3 = binf == 3
    sel_s = jnp.where(b1, chan(19), jnp.where(b2, chan(21),
                      jnp.where(b3, chan(23), chan(17))))
    sel_c = jnp.where(b1, chan(20), jnp.where(b2, chan(22),
                      jnp.where(b3, chan(24), chan(18))))
    nsq = jnp.maximum(sel_s * sel_s + sel_c * sel_c, 1e-30)
    nrm = nsq * _rsqrt(nsq) + 1e-9
    ac = jnp.where(b1, _PI / 2.0, jnp.where(b2, _PI,
                   jnp.where(b3, -_PI / 2.0, 0.0)))
    alpha = _wrapf(_atan2(sel_s / nrm, sel_c / nrm) + ac)
    roty = _wrapf(alpha + _atan2(x3d, depth))

    outs = (clsf, alpha, x1, y1, x2, y2, d0, d1, d2, x3d, y3d, depth,
            roty, score)
    for c, vec in enumerate(outs):
        plsc.store_scatter(obuf, [lane8 * 16 + c], vec * valid)
    pltpu.sync_copy(obuf, out_hbm.at[pl.ds((b * 64 + k0) * 16, 128)])


_sc_call = pl.kernel(
    _sc_body,
    out_type=jax.ShapeDtypeStruct((4096,), jnp.float32),
    mesh=plsc.VectorSubcoreMesh(core_axis_name="c", subcore_axis_name="s"),
    compiler_params=pltpu.CompilerParams(
        needs_layout_passes=False, use_tc_tiling_on_sc=False),
    scratch_types=[
        pltpu.VMEM((40 * 320,), jnp.float32),
        pltpu.VMEM((352,), jnp.float32),
        pltpu.VMEM((36 * 320,), jnp.float32),
        pltpu.VMEM((720,), jnp.float32),
        pltpu.VMEM((48,), jnp.float32),
        pltpu.VMEM((128,), jnp.float32),
        pltpu.VMEM((128,), jnp.int32),
        pltpu.VMEM((1024,), jnp.float32),
        pltpu.VMEM((1024,), jnp.int32),
        pltpu.VMEM((64,), jnp.float32),
        pltpu.VMEM((128,), jnp.float32),
        pltpu.VMEM((128,), jnp.int32),
        pltpu.VMEM((128,), jnp.float32),
        pltpu.VMEM((128,), jnp.int32),
        pltpu.VMEM((4, 128), jnp.int32),
        pltpu.VMEM((512, 16), jnp.float32),
        pltpu.VMEM((128,), jnp.float32),
        pltpu.VMEM_SHARED((2048,), jnp.float32),
        pltpu.VMEM_SHARED((2048,), jnp.int32),
        pltpu.VMEM_SHARED((256,), jnp.float32),
        pltpu.VMEM_SHARED((256,), jnp.int32),
        pltpu.SemaphoreType.DMA,
    ],
)


def _atan2(y, x):
    ax = jnp.abs(x)
    ay = jnp.abs(y)
    swap = ay > ax
    num = jnp.where(swap, ax, ay)
    den = jnp.where(swap, ay, ax)
    t = num / jnp.maximum(den, 1e-30)
    red = t > 0.4142135623730950488
    z = jnp.where(red, (t - 1.0) / (t + 1.0), t)
    z2 = z * z
    pp = ((8.05374449538e-2 * z2 - 1.38776856032e-1) * z2
          + 1.99777106478e-1) * z2 - 3.33329491539e-1
    r = z + z * z2 * pp
    r = jnp.where(red, r + 0.7853981633974483, r)
    r = jnp.where(swap, 1.5707963267948966 - r, r)
    r = jnp.where(x < 0.0, _PI - r, r)
    return jnp.where(y < 0.0, -r, r)


def _rsqrt(x):
    i = plsc.bitcast(x, jnp.int32)
    y = plsc.bitcast(jnp.int32(0x5F3759DF) - (i >> 1), jnp.float32)
    for _ in range(3):
        y = y * (1.5 - 0.5 * x * y * y)
    return y


def _floorf(x):
    t = x.astype(jnp.int32).astype(jnp.float32)
    return t - jnp.where(x < t, 1.0, 0.0)


def _wrapf(a):
    m = a + _PI
    m = m - (2.0 * _PI) * _floorf(m / (2.0 * _PI))
    return m - _PI


def kernel(pred_heatmap, pred_regression):
    heat1 = pred_heatmap.reshape(-1)
    reg2 = pred_regression.reshape(_B * 50 * (_HW // 16), 16)
    out = _sc_call(heat1, reg2)
    return out.reshape(_B, 64, 16)[:, :_K, :14].reshape(_B * _K, 14)
